# baseline (speedup 1.0000x reference)
import functools

import jax
import jax.numpy as jnp
from jax import lax
from jax.experimental import pallas as pl
from jax.experimental.pallas import tpu as pltpu
from jax.experimental.pallas import tpu_sc as plsc

N = 10000
E = 320000
NP = 10240
PADNODE = 10200
NC = 2
NS = 16
NW = NC * NS
CH = 128
NCHUNK = 80
EP = NW * NCHUNK * CH
NCHUNK2 = EP // (NS * CH)
RPT = NP // NS


def _zero_f32_rows(ref, rows, cols):
  zv = jnp.zeros((16,), jnp.float32)
  for r in range(rows):
    for j in range(cols // 16):
      ref[r, pl.ds(j * 16, 16)] = zv


_sc_mesh = plsc.VectorSubcoreMesh(core_axis_name="c", subcore_axis_name="s")
_sc_params = pltpu.CompilerParams(
    use_tc_tiling_on_sc=False,
    disable_bounds_checks=True,
    disable_semaphore_checks=True,
)


@functools.partial(
    pl.kernel,
    out_type=jax.ShapeDtypeStruct((NC, NP), jnp.float32),
    mesh=_sc_mesh,
    compiler_params=_sc_params,
    scratch_types=[
        pltpu.VMEM((CH,), jnp.float32),
        pltpu.VMEM((NCHUNK, CH), jnp.int32),
        pltpu.VMEM((RPT,), jnp.float32),
        pltpu.VMEM_SHARED((NP,), jnp.float32),
        pltpu.SemaphoreType.DMA((8,)),
    ],
)
def _sc_degree(dst_hbm, out_hbm, ones_v, didx_v, tmp_v, dacc, sems):
  c = lax.axis_index("c")
  s = lax.axis_index("s")
  w = s * NC + c

  one = jnp.full((16,), 1.0, jnp.float32)
  zero = jnp.zeros((16,), jnp.float32)
  for j in range(CH // 16):
    ones_v[pl.ds(j * 16, 16)] = one
  for j in range(RPT // 16):
    tmp_v[pl.ds(j * 16, 16)] = zero

  pltpu.sync_copy(dst_hbm.at[w], didx_v)
  pltpu.sync_copy(tmp_v, dacc.at[pl.ds(s * RPT, RPT)])
  plsc.subcore_barrier()

  @pl.loop(0, NCHUNK // 8)
  def _(k):
    base = k * 8
    descs = []
    for j in range(8):
      descs.append(
          pltpu.async_copy(ones_v, dacc.at[didx_v.at[base + j]],
                           sems.at[j], add=True))
    for d in descs:
      d.wait()

  plsc.subcore_barrier()
  pltpu.sync_copy(dacc.at[pl.ds(s * RPT, RPT)], tmp_v)
  pltpu.sync_copy(tmp_v, out_hbm.at[c, pl.ds(s * RPT, RPT)])


def _make_sc_scatter(D, edge_split, nbuf):
  nchunk = NCHUNK if edge_split else NCHUNK2

  @functools.partial(
      pl.kernel,
      out_type=jax.ShapeDtypeStruct((NC, NP, D), jnp.float32),
      mesh=_sc_mesh,
      compiler_params=_sc_params,
      scratch_types=[
          pltpu.VMEM((nchunk, CH), jnp.int32),
          pltpu.VMEM((nchunk, CH), jnp.int32),
          pltpu.VMEM((2 * nbuf, CH, D), jnp.float32),
          pltpu.VMEM((16, D), jnp.float32),
          pltpu.VMEM((RPT, D), jnp.float32),
          pltpu.VMEM_SHARED((NP, D), jnp.float32),
          pltpu.SemaphoreType.DMA((nbuf,)),
          pltpu.SemaphoreType.DMA((nbuf,)),
          pltpu.SemaphoreType.DMA((nbuf,)),
          pltpu.SemaphoreType.DMA((nbuf,)),
          pltpu.SemaphoreType.DMA,
      ],
  )
  def sc_scatter(src_hbm, dst_hbm, p_hbm, out_hbm,
                 sidx_v, didx_v, rows_v, zsrc_v, tmp_v, acc,
                 gA, gB, sA, sB, isem):
    c = lax.axis_index("c")
    s = lax.axis_index("s")
    slot = s * NC + c if edge_split else s

    i1 = pltpu.async_copy(src_hbm.at[slot], sidx_v, isem)
    i2 = pltpu.async_copy(dst_hbm.at[slot], didx_v, isem)

    _zero_f32_rows(zsrc_v, 16, D)

    @pl.loop(0, RPT // 16)
    def _(i):
      pltpu.sync_copy(zsrc_v, acc.at[pl.ds(s * RPT + i * 16, 16)])

    i1.wait()
    i2.wait()
    plsc.subcore_barrier()

    psrc = p_hbm if edge_split else p_hbm.at[c]

    def issue_gathers(base, boff, sems):
      for j in range(nbuf):
        pltpu.async_copy(psrc.at[sidx_v.at[base + j]], rows_v.at[boff + j],
                         sems.at[j])

    def drain(sems, boff):
      for j in range(nbuf):
        pltpu.make_async_copy(psrc.at[pl.ds(0, CH)], rows_v.at[boff + j],
                              sems.at[j]).wait()

    def scatter_round(base, boff, gsems, ssems):
      for j in range(nbuf):
        pltpu.make_async_copy(psrc.at[pl.ds(0, CH)], rows_v.at[boff + j],
                              gsems.at[j]).wait()
        pltpu.async_copy(rows_v.at[boff + j], acc.at[didx_v.at[base + j]],
                         ssems.at[j], add=True)

    nrounds = nchunk // nbuf

    issue_gathers(0, 0, gA)

    @pl.loop(0, nrounds // 2)
    def _(k2):
      r0 = 2 * k2
      scatter_round(r0 * nbuf, 0, gA, sA)
      @pl.when(k2 > 0)
      def _():
        drain(sB, nbuf)
      issue_gathers((r0 + 1) * nbuf, nbuf, gB)
      scatter_round((r0 + 1) * nbuf, nbuf, gB, sB)
      drain(sA, 0)
      @pl.when(k2 < nrounds // 2 - 1)
      def _():
        issue_gathers((r0 + 2) * nbuf, 0, gA)

    drain(sB, nbuf)

    plsc.subcore_barrier()
    pltpu.sync_copy(acc.at[pl.ds(s * RPT, RPT)], tmp_v)
    pltpu.sync_copy(tmp_v, out_hbm.at[c, pl.ds(s * RPT, RPT)])

  return sc_scatter


_sc_scatter_l1 = _make_sc_scatter(32, False, 5)
_sc_scatter_l2 = _make_sc_scatter(16, False, 5)


def _tc1_body(dega, degb, x, w1, h1p, dinv):
  deg = dega[...] + degb[...] + 1.0
  di = lax.rsqrt(deg)
  xw = jnp.dot(x[...], w1[...], preferred_element_type=jnp.float32)
  xwp = jnp.concatenate(
      [xw, jnp.zeros((NP - N, 64), jnp.float32)], axis=0)
  p = xwp * di
  dinv[...] = di
  h1p[0] = p[:, :32]
  h1p[1] = p[:, 32:]


def _tc2_body(agg, h1p, dinv, b1, w2, h2p):
  di = dinv[...]
  tot = jnp.concatenate([agg[0] + h1p[0], agg[1] + h1p[1]], axis=1)
  h = jnp.maximum(di * tot + b1[...], 0.0)
  hw = jnp.dot(h, w2[...], preferred_element_type=jnp.float32) * di
  h2p[0] = hw[:, :16]
  h2p[1] = hw[:, 16:]


def _tc3_body(agg, h2p, dinv, b2, wfc, bfc, out):
  di = dinv[...]
  tot = jnp.concatenate([agg[0] + h2p[0], agg[1] + h2p[1]], axis=1)
  z = (di * tot + b2[...])[:N]
  logits = jnp.sum(z * wfc[...], axis=1, keepdims=True) + bfc[...]
  out[...] = jax.nn.sigmoid(logits)


_tc1 = pl.pallas_call(
    _tc1_body,
    out_shape=[jax.ShapeDtypeStruct((NC, NP, 32), jnp.float32),
               jax.ShapeDtypeStruct((NP, 1), jnp.float32)],
)

_tc2 = pl.pallas_call(
    _tc2_body,
    out_shape=jax.ShapeDtypeStruct((NC, NP, 16), jnp.float32),
)

_tc3 = pl.pallas_call(
    _tc3_body,
    out_shape=jax.ShapeDtypeStruct((N, 1), jnp.float32),
)


@jax.jit
def _run(x, edge_index, W1, b1, W2, b2, W_fc, b_fc):
  ei = edge_index.astype(jnp.int32)
  pad = jnp.full((EP - E,), PADNODE, jnp.int32)
  srcf = jnp.concatenate([ei[0], pad])
  dstf = jnp.concatenate([ei[1], pad])
  src_w = srcf.reshape(NW, NCHUNK, CH)
  dst_w = dstf.reshape(NW, NCHUNK, CH)
  src_t = srcf.reshape(NS, NCHUNK2, CH)
  dst_t = dstf.reshape(NS, NCHUNK2, CH)

  deg2 = _sc_degree(dst_w)
  dega = deg2[0].reshape(NP, 1)
  degb = deg2[1].reshape(NP, 1)

  h1p, dinv = _tc1(dega, degb, x, W1)
  agg1 = _sc_scatter_l1(src_t, dst_t, h1p)
  h2p = _tc2(agg1, h1p, dinv, b1.reshape(1, 64), W2)
  agg2 = _sc_scatter_l2(src_t, dst_t, h2p)
  return _tc3(agg2, h2p, dinv, b2.reshape(1, 32),
              W_fc.reshape(1, 32), b_fc.reshape(1, 1))


def kernel(x, edge_index, W1, b1, W2, b2, W_fc, b_fc):
  return _run(x, edge_index, W1, b1, W2, b2, W_fc, b_fc)


# speedup vs baseline: 1.0580x; 1.0580x over previous
import functools

import jax
import jax.numpy as jnp
from jax import lax
from jax.experimental import pallas as pl
from jax.experimental.pallas import tpu as pltpu
from jax.experimental.pallas import tpu_sc as plsc

N = 10000
E = 320000
NP = 10240
PADNODE = 10200
NC = 2
NS = 16
NW = NC * NS
CH = 128
NCHUNK = 80
EP = NW * NCHUNK * CH
NCHUNK2 = EP // (NS * CH)
RPT = NP // NS


def _zero_f32_rows(ref, rows, cols):
  zv = jnp.zeros((16,), jnp.float32)
  for r in range(rows):
    for j in range(cols // 16):
      ref[r, pl.ds(j * 16, 16)] = zv


_sc_mesh = plsc.VectorSubcoreMesh(core_axis_name="c", subcore_axis_name="s")
_sc_params = pltpu.CompilerParams(use_tc_tiling_on_sc=False)


@functools.partial(
    pl.kernel,
    out_type=jax.ShapeDtypeStruct((NC, NP), jnp.float32),
    mesh=_sc_mesh,
    compiler_params=_sc_params,
    scratch_types=[
        pltpu.VMEM((CH,), jnp.float32),
        pltpu.VMEM((NCHUNK, CH), jnp.int32),
        pltpu.VMEM((RPT,), jnp.float32),
        pltpu.VMEM_SHARED((NP,), jnp.float32),
        pltpu.SemaphoreType.DMA((8,)),
    ],
)
def _sc_degree(dst_hbm, out_hbm, ones_v, didx_v, tmp_v, dacc, sems):
  c = lax.axis_index("c")
  s = lax.axis_index("s")
  w = s * NC + c

  one = jnp.full((16,), 1.0, jnp.float32)
  zero = jnp.zeros((16,), jnp.float32)
  for j in range(CH // 16):
    ones_v[pl.ds(j * 16, 16)] = one
  for j in range(RPT // 16):
    tmp_v[pl.ds(j * 16, 16)] = zero

  pltpu.sync_copy(dst_hbm.at[w], didx_v)
  pltpu.sync_copy(tmp_v, dacc.at[pl.ds(s * RPT, RPT)])
  plsc.subcore_barrier()

  @pl.loop(0, NCHUNK // 8)
  def _(k):
    base = k * 8
    descs = []
    for j in range(8):
      descs.append(
          pltpu.async_copy(ones_v, dacc.at[didx_v.at[base + j]],
                           sems.at[j], add=True))
    for d in descs:
      d.wait()

  plsc.subcore_barrier()
  pltpu.sync_copy(dacc.at[pl.ds(s * RPT, RPT)], tmp_v)
  pltpu.sync_copy(tmp_v, out_hbm.at[c, pl.ds(s * RPT, RPT)])


def _make_sc_scatter(D, edge_split, nbuf):
  nchunk = NCHUNK if edge_split else NCHUNK2

  @functools.partial(
      pl.kernel,
      out_type=jax.ShapeDtypeStruct((NC, NP, D), jnp.float32),
      mesh=_sc_mesh,
      compiler_params=_sc_params,
      scratch_types=[
          pltpu.VMEM((nchunk, CH), jnp.int32),
          pltpu.VMEM((nchunk, CH), jnp.int32),
          pltpu.VMEM((2 * nbuf, CH, D), jnp.float32),
          pltpu.VMEM((16, D), jnp.float32),
          pltpu.VMEM((RPT, D), jnp.float32),
          pltpu.VMEM_SHARED((NP, D), jnp.float32),
          pltpu.SemaphoreType.DMA((nbuf,)),
          pltpu.SemaphoreType.DMA((nbuf,)),
          pltpu.SemaphoreType.DMA((nbuf,)),
          pltpu.SemaphoreType.DMA((nbuf,)),
          pltpu.SemaphoreType.DMA,
      ],
  )
  def sc_scatter(src_hbm, dst_hbm, p_hbm, out_hbm,
                 sidx_v, didx_v, rows_v, zsrc_v, tmp_v, acc,
                 gA, gB, sA, sB, isem):
    c = lax.axis_index("c")
    s = lax.axis_index("s")
    slot = s * NC + c if edge_split else s

    i1 = pltpu.async_copy(src_hbm.at[slot], sidx_v, isem)
    i2 = pltpu.async_copy(dst_hbm.at[slot], didx_v, isem)

    _zero_f32_rows(zsrc_v, 16, D)

    @pl.loop(0, RPT // 16)
    def _(i):
      pltpu.sync_copy(zsrc_v, acc.at[pl.ds(s * RPT + i * 16, 16)])

    i1.wait()
    i2.wait()
    plsc.subcore_barrier()

    psrc = p_hbm if edge_split else p_hbm.at[c]

    def issue_gathers(base, boff, sems):
      for j in range(nbuf):
        pltpu.async_copy(psrc.at[sidx_v.at[base + j]], rows_v.at[boff + j],
                         sems.at[j])

    def drain(sems, boff):
      for j in range(nbuf):
        pltpu.make_async_copy(psrc.at[pl.ds(0, CH)], rows_v.at[boff + j],
                              sems.at[j]).wait()

    def scatter_round(base, boff, gsems, ssems):
      for j in range(nbuf):
        pltpu.make_async_copy(psrc.at[pl.ds(0, CH)], rows_v.at[boff + j],
                              gsems.at[j]).wait()
        pltpu.async_copy(rows_v.at[boff + j], acc.at[didx_v.at[base + j]],
                         ssems.at[j], add=True)

    nrounds = nchunk // nbuf

    issue_gathers(0, 0, gA)

    @pl.loop(0, nrounds // 2)
    def _(k2):
      r0 = 2 * k2
      scatter_round(r0 * nbuf, 0, gA, sA)
      @pl.when(k2 > 0)
      def _():
        drain(sB, nbuf)
      issue_gathers((r0 + 1) * nbuf, nbuf, gB)
      scatter_round((r0 + 1) * nbuf, nbuf, gB, sB)
      drain(sA, 0)
      @pl.when(k2 < nrounds // 2 - 1)
      def _():
        issue_gathers((r0 + 2) * nbuf, 0, gA)

    drain(sB, nbuf)

    plsc.subcore_barrier()
    pltpu.sync_copy(acc.at[pl.ds(s * RPT, RPT)], tmp_v)
    pltpu.sync_copy(tmp_v, out_hbm.at[c, pl.ds(s * RPT, RPT)])

  return sc_scatter


_sc_scatter_l1 = _make_sc_scatter(32, False, 5)
_sc_scatter_l2 = _make_sc_scatter(16, False, 5)


def _tc1_body(dega, degb, x, w1, h1p, dinv):
  deg = dega[...] + degb[...] + 1.0
  di = lax.rsqrt(deg)
  xw = jnp.dot(x[...], w1[...], preferred_element_type=jnp.float32)
  xwp = jnp.concatenate(
      [xw, jnp.zeros((NP - N, 64), jnp.float32)], axis=0)
  p = xwp * di
  dinv[...] = di
  h1p[0] = p[:, :32]
  h1p[1] = p[:, 32:]


def _tc2_body(agg, h1p, dinv, b1, w2, h2p):
  di = dinv[...]
  tot = jnp.concatenate([agg[0] + h1p[0], agg[1] + h1p[1]], axis=1)
  h = jnp.maximum(di * tot + b1[...], 0.0)
  hw = jnp.dot(h, w2[...], preferred_element_type=jnp.float32) * di
  h2p[0] = hw[:, :16]
  h2p[1] = hw[:, 16:]


def _tc3_body(agg, h2p, dinv, b2, wfc, bfc, out):
  di = dinv[...]
  tot = jnp.concatenate([agg[0] + h2p[0], agg[1] + h2p[1]], axis=1)
  z = di * tot + b2[...]
  logits = jnp.sum(z * wfc[...], axis=1, keepdims=True) + bfc[...]
  out[...] = jax.nn.sigmoid(logits)


_tc1 = pl.pallas_call(
    _tc1_body,
    out_shape=[jax.ShapeDtypeStruct((NC, NP, 32), jnp.float32),
               jax.ShapeDtypeStruct((NP, 1), jnp.float32)],
)

_tc2 = pl.pallas_call(
    _tc2_body,
    out_shape=jax.ShapeDtypeStruct((NC, NP, 16), jnp.float32),
)

_tc3 = pl.pallas_call(
    _tc3_body,
    out_shape=jax.ShapeDtypeStruct((NP, 1), jnp.float32),
)


@jax.jit
def _run(x, edge_index, W1, b1, W2, b2, W_fc, b_fc):
  ei = edge_index.astype(jnp.int32)
  pad = jnp.full((EP - E,), PADNODE, jnp.int32)
  srcf = jnp.concatenate([ei[0], pad])
  dstf = jnp.concatenate([ei[1], pad])
  src_w = srcf.reshape(NW, NCHUNK, CH)
  dst_w = dstf.reshape(NW, NCHUNK, CH)
  src_t = srcf.reshape(NS, NCHUNK2, CH)
  dst_t = dstf.reshape(NS, NCHUNK2, CH)

  deg2 = _sc_degree(dst_w)
  dega = deg2[0].reshape(NP, 1)
  degb = deg2[1].reshape(NP, 1)

  h1p, dinv = _tc1(dega, degb, x, W1)
  agg1 = _sc_scatter_l1(src_t, dst_t, h1p)
  h2p = _tc2(agg1, h1p, dinv, b1.reshape(1, 64), W2)
  agg2 = _sc_scatter_l2(src_t, dst_t, h2p)
  out = _tc3(agg2, h2p, dinv, b2.reshape(1, 32),
             W_fc.reshape(1, 32), b_fc.reshape(1, 1))
  return out[:N]


def kernel(x, edge_index, W1, b1, W2, b2, W_fc, b_fc):
  return _run(x, edge_index, W1, b1, W2, b2, W_fc, b_fc)
